# baseline (speedup 1.0000x reference)
import functools

import jax
import jax.numpy as jnp
from jax import lax
from jax.experimental import pallas as pl
from jax.experimental.pallas import tpu as pltpu
from jax.experimental.pallas import tpu_sc as plsc

NUM_ROWS = 1000000
NUM_FACTORS = 32
BATCH = 16384
NC = 2
NS = 16
NW = NC * NS
BPW = BATCH // NW
NSLOT = 8
GRP = 16
NGRP = BPW // GRP

_mesh = plsc.VectorSubcoreMesh(
    core_axis_name="c", subcore_axis_name="s", num_cores=NC, num_subcores=NS
)


@functools.partial(
    pl.kernel,
    out_type=jax.ShapeDtypeStruct((BATCH,), jnp.float32),
    mesh=_mesh,
    compiler_params=pltpu.CompilerParams(needs_layout_passes=False),
    scratch_types=[
        pltpu.VMEM((BPW,), jnp.int32),
        pltpu.VMEM((BPW,), jnp.int32),
        pltpu.VMEM((NSLOT * NUM_FACTORS, 128), jnp.float32),
        pltpu.VMEM((NSLOT * NUM_FACTORS, 128), jnp.float32),
        pltpu.VMEM((GRP, 16), jnp.float32),
        pltpu.VMEM((BPW,), jnp.float32),
    ]
    + [pltpu.SemaphoreType.DMA] * (2 * NSLOT),
)
def _mf_dot(uf_t, if_t, user_hbm, item_hbm, out_hbm,
            us_s, is_s, ubuf, ibuf, part_v, out_v, *sems):
    usem, isem = sems[:NSLOT], sems[NSLOT:]
    wid = lax.axis_index("s") * NC + lax.axis_index("c")
    base = wid * BPW

    pltpu.sync_copy(user_hbm.at[pl.ds(base, BPW)], us_s)
    pltpu.sync_copy(item_hbm.at[pl.ds(base, BPW)], is_s)

    iota16 = lax.iota(jnp.int32, 16)

    def fire(ru, ri, slot):
        uc0 = pl.multiple_of((ru >> 7) << 7, 128)
        ic0 = pl.multiple_of((ri >> 7) << 7, 128)
        for p in range(NUM_FACTORS // 8):
            pltpu.async_copy(
                uf_t.at[pl.ds(8 * p, 8), pl.ds(uc0, 128)],
                ubuf.at[pl.ds(slot * NUM_FACTORS + 8 * p, 8)], usem[slot])
            pltpu.async_copy(
                if_t.at[pl.ds(8 * p, 8), pl.ds(ic0, 128)],
                ibuf.at[pl.ds(slot * NUM_FACTORS + 8 * p, 8)], isem[slot])

    def drain(slot):
        pltpu.make_async_copy(
            uf_t.at[:, pl.ds(0, 128)],
            ubuf.at[pl.ds(slot * NUM_FACTORS, NUM_FACTORS)], usem[slot]).wait()
        pltpu.make_async_copy(
            if_t.at[:, pl.ds(0, 128)],
            ibuf.at[pl.ds(slot * NUM_FACTORS, NUM_FACTORS)], isem[slot]).wait()

    def item(uvec, ivec, j, slot, prow):
        drain(slot)
        ulane = jnp.full((16,), uvec[j] & 127, jnp.int32)
        ilane = jnp.full((16,), ivec[j] & 127, jnp.int32)
        rbase = slot * NUM_FACTORS
        u_lo = plsc.load_gather(ubuf, [rbase + iota16, ulane])
        u_hi = plsc.load_gather(ubuf, [rbase + 16 + iota16, ulane])
        i_lo = plsc.load_gather(ibuf, [rbase + iota16, ilane])
        i_hi = plsc.load_gather(ibuf, [rbase + 16 + iota16, ilane])
        part_v[prow] = u_lo * i_lo + u_hi * i_hi

    uvec0 = us_s[pl.ds(0, 16)]
    ivec0 = is_s[pl.ds(0, 16)]
    for j in range(NSLOT):
        fire(uvec0[j], ivec0[j], j)

    def group(g, _):
        uvec = us_s[pl.ds(pl.multiple_of(g * GRP, 16), 16)]
        ivec = is_s[pl.ds(pl.multiple_of(g * GRP, 16), 16)]
        nxt = pl.multiple_of((g + 1) * GRP, 16)
        nxt = pl.multiple_of(jnp.minimum(nxt, BPW - 16), 16)
        uvecn = us_s[pl.ds(nxt, 16)]
        ivecn = is_s[pl.ds(nxt, 16)]

        for j in range(NSLOT):
            item(uvec, ivec, j, j, j)
            fire(uvec[NSLOT + j], ivec[NSLOT + j], j)
        for j in range(NSLOT):
            item(uvec, ivec, NSLOT + j, j, NSLOT + j)

            @pl.when(g + 1 < NGRP)
            def _(j=j):
                fire(uvecn[j], ivecn[j], j)

        acc = jnp.zeros((16,), jnp.float32)
        for f in range(16):
            acc = acc + plsc.load_gather(
                part_v, [iota16, jnp.full((16,), f, jnp.int32)])
        out_v[pl.ds(pl.multiple_of(g * GRP, 16), GRP)] = acc
        return 0

    lax.fori_loop(0, NGRP, group, 0)
    pltpu.sync_copy(out_v, out_hbm.at[pl.ds(base, BPW)])


def kernel(user_factors, item_factors, user, item):
    return _mf_dot(user_factors.T, item_factors.T,
                   user.astype(jnp.int32), item.astype(jnp.int32))


# speedup vs baseline: 1.0030x; 1.0030x over previous
import functools

import jax
import jax.numpy as jnp
from jax import lax
from jax.experimental import pallas as pl
from jax.experimental.pallas import tpu as pltpu
from jax.experimental.pallas import tpu_sc as plsc

NUM_ROWS = 1000000
NUM_FACTORS = 32
BATCH = 16384
NC = 2
NS = 16
NW = NC * NS
BPW = BATCH // NW
NSLOT = 8
GRP = 16
NGRP = BPW // GRP

_mesh = plsc.VectorSubcoreMesh(
    core_axis_name="c", subcore_axis_name="s", num_cores=NC, num_subcores=NS
)


@functools.partial(
    pl.kernel,
    out_type=jax.ShapeDtypeStruct((BATCH,), jnp.float32),
    mesh=_mesh,
    compiler_params=pltpu.CompilerParams(needs_layout_passes=False),
    scratch_types=[
        pltpu.VMEM((BPW,), jnp.int32),
        pltpu.VMEM((BPW,), jnp.int32),
        pltpu.VMEM((NSLOT * NUM_FACTORS, 128), jnp.float32),
        pltpu.VMEM((NSLOT * NUM_FACTORS, 128), jnp.float32),
        pltpu.VMEM((GRP, 16), jnp.float32),
        pltpu.VMEM((BPW,), jnp.float32),
    ]
    + [pltpu.SemaphoreType.DMA] * (2 * NSLOT),
)
def _mf_dot(uf_t, if_t, user_hbm, item_hbm, out_hbm,
            us_s, is_s, ubuf, ibuf, part_v, out_v, *sems):
    usem, isem = sems[:NSLOT], sems[NSLOT:]
    wid = lax.axis_index("s") * NC + lax.axis_index("c")
    base = wid * BPW

    pltpu.sync_copy(user_hbm.at[pl.ds(base, BPW)], us_s)
    pltpu.sync_copy(item_hbm.at[pl.ds(base, BPW)], is_s)

    iota16 = lax.iota(jnp.int32, 16)

    def fire(ru, ri, slot):
        uc0 = pl.multiple_of((ru >> 7) << 7, 128)
        ic0 = pl.multiple_of((ri >> 7) << 7, 128)
        pltpu.async_copy(
            uf_t.at[:, pl.ds(uc0, 128)],
            ubuf.at[pl.ds(slot * NUM_FACTORS, NUM_FACTORS)], usem[slot])
        pltpu.async_copy(
            if_t.at[:, pl.ds(ic0, 128)],
            ibuf.at[pl.ds(slot * NUM_FACTORS, NUM_FACTORS)], isem[slot])

    def drain(slot):
        pltpu.make_async_copy(
            uf_t.at[:, pl.ds(0, 128)],
            ubuf.at[pl.ds(slot * NUM_FACTORS, NUM_FACTORS)], usem[slot]).wait()
        pltpu.make_async_copy(
            if_t.at[:, pl.ds(0, 128)],
            ibuf.at[pl.ds(slot * NUM_FACTORS, NUM_FACTORS)], isem[slot]).wait()

    def item(uvec, ivec, j, slot, prow):
        drain(slot)
        ulane = jnp.full((16,), uvec[j] & 127, jnp.int32)
        ilane = jnp.full((16,), ivec[j] & 127, jnp.int32)
        rbase = slot * NUM_FACTORS
        u_lo = plsc.load_gather(ubuf, [rbase + iota16, ulane])
        u_hi = plsc.load_gather(ubuf, [rbase + 16 + iota16, ulane])
        i_lo = plsc.load_gather(ibuf, [rbase + iota16, ilane])
        i_hi = plsc.load_gather(ibuf, [rbase + 16 + iota16, ilane])
        part_v[prow] = u_lo * i_lo + u_hi * i_hi

    uvec0 = us_s[pl.ds(0, 16)]
    ivec0 = is_s[pl.ds(0, 16)]
    for j in range(NSLOT):
        fire(uvec0[j], ivec0[j], j)

    def group(g, _):
        uvec = us_s[pl.ds(pl.multiple_of(g * GRP, 16), 16)]
        ivec = is_s[pl.ds(pl.multiple_of(g * GRP, 16), 16)]
        nxt = pl.multiple_of((g + 1) * GRP, 16)
        nxt = pl.multiple_of(jnp.minimum(nxt, BPW - 16), 16)
        uvecn = us_s[pl.ds(nxt, 16)]
        ivecn = is_s[pl.ds(nxt, 16)]

        for j in range(NSLOT):
            item(uvec, ivec, j, j, j)
            fire(uvec[NSLOT + j], ivec[NSLOT + j], j)
        for j in range(NSLOT):
            item(uvec, ivec, NSLOT + j, j, NSLOT + j)

            @pl.when(g + 1 < NGRP)
            def _(j=j):
                fire(uvecn[j], ivecn[j], j)

        acc = jnp.zeros((16,), jnp.float32)
        for f in range(16):
            acc = acc + plsc.load_gather(
                part_v, [iota16, jnp.full((16,), f, jnp.int32)])
        out_v[pl.ds(pl.multiple_of(g * GRP, 16), GRP)] = acc
        return 0

    lax.fori_loop(0, NGRP, group, 0)
    pltpu.sync_copy(out_v, out_hbm.at[pl.ds(base, BPW)])


def kernel(user_factors, item_factors, user, item):
    return _mf_dot(user_factors.T, item_factors.T,
                   user.astype(jnp.int32), item.astype(jnp.int32))
